# baseline (speedup 1.0000x reference)
import functools
import math

import jax
import jax.numpy as jnp
from jax import lax
from jax.experimental import pallas as pl
from jax.experimental.pallas import tpu as pltpu
from jax.experimental.pallas import tpu_sc as plsc

_B_SC = 8


def _qk_body(q_ref, wq_ref, bq_ref, wk_ref, qk_ref):
    qwq = lax.dot_general(q_ref[...], wq_ref[...],
                          (((1,), (1,)), ((), ())),
                          preferred_element_type=jnp.float32)
    Q = jnp.tanh(qwq + bq_ref[...])
    qk_ref[...] = lax.dot_general(Q, wk_ref[...],
                                  (((1,), (0,)), ((), ())),
                                  preferred_element_type=jnp.float32)


def _attn_body(qk_ref, wv_ref, k_ref, v_ref, out_ref):
    kb = k_ref[0]
    vb = v_ref[0]
    qk = qk_ref[0]
    D = kb.shape[-1]
    M = kb.shape[1]
    scale = 1.0 / math.sqrt(D)

    att = jnp.sum(kb * qk[None, :, :], axis=2) * scale
    att = att.T
    att = jnp.where(att == 0.0, jnp.float32(-10000.0), att)
    att = jnp.where(att >= 0.0, att, 0.01 * att)
    amax = jnp.max(att, axis=0, keepdims=True)
    e = jnp.exp(att - amax)
    p = e / jnp.sum(e, axis=0, keepdims=True)
    p = jnp.where(p == jnp.float32(1.0 / M), jnp.float32(0.0), p)

    ctx = lax.dot_general(p, vb, (((0,), (1,)), ((1,), (0,))),
                          preferred_element_type=jnp.float32)
    out_ref[0] = lax.dot_general(ctx, wv_ref[...],
                                 (((1,), (1,)), ((), ())),
                                 preferred_element_type=jnp.float32)


def _sc_proj_body(ctx_ref, wv_ref, out_ref):
    out_ref[...] = lax.dot_general(ctx_ref[...], wv_ref[...],
                                   (((1,), (1,)), ((), ())),
                                   preferred_element_type=jnp.float32)


def _make_sc_ctx(B, N, M, D, b_start):
    B_SC = B - b_start
    R = B_SC * N
    MD = M * D
    W = 32
    RPW = R // W
    WPB = N // RPW
    L = 16
    NC = D // L
    scale = 1.0 / math.sqrt(D)
    mesh = plsc.VectorSubcoreMesh(core_axis_name="c", subcore_axis_name="s")

    @functools.partial(
        pl.kernel,
        out_type=jax.ShapeDtypeStruct((R * D,), jnp.float32),
        mesh=mesh,
        scratch_types=[
            pltpu.VMEM((D,), jnp.float32),
            pltpu.VMEM((MD,), jnp.float32),
            pltpu.VMEM((MD,), jnp.float32),
            pltpu.VMEM((2 * L,), jnp.float32),
            pltpu.VMEM((D,), jnp.float32),
            pltpu.SemaphoreType.DMA,
            pltpu.SemaphoreType.DMA,
        ],
    )
    def sc_ctx(k_hbm, v_hbm, qk_hbm, out_hbm,
               qk_v, k_v, v_v, red_v, ctx_v, sem_k, sem_v):
        wid = lax.axis_index("s") * 2 + lax.axis_index("c")
        b_loc = wid // WPB
        base_loc = b_loc * N + (wid % WPB) * RPW
        pltpu.sync_copy(qk_hbm.at[pl.ds((b_start + b_loc) * D, D)], qk_v)
        qkc = [qk_v[pl.ds(i * L, L)] for i in range(NC)]
        lane = lax.iota(jnp.int32, L)
        zeros = jnp.zeros((L,), jnp.float32)

        def _lane_sum(x):
            red_v[pl.ds(L, L)] = zeros
            for sh in (8, 4, 2, 1):
                red_v[pl.ds(0, L)] = x
                x = x + red_v[pl.ds(sh, L)]
            return x[0]

        def row_body(j, carry):
            r_loc = base_loc + j
            off = (b_start * N + r_loc) * MD
            ck = pltpu.async_copy(k_hbm.at[pl.ds(off, MD)], k_v, sem_k)
            cv = pltpu.async_copy(v_hbm.at[pl.ds(off, MD)], v_v, sem_v)
            ck.wait()
            cv.wait()

            a0 = zeros
            a1 = zeros
            for m in range(M):
                acc = k_v[pl.ds(m * D, L)] * qkc[0]
                for i in range(1, NC):
                    acc = acc + k_v[pl.ds(m * D + i * L, L)] * qkc[i]
                tot = jnp.full((L,), _lane_sum(acc) * scale, jnp.float32)
                if m < L:
                    a0 = jnp.where(lane == m, tot, a0)
                else:
                    a1 = jnp.where(lane == (m - L), tot, a1)

            neg = jnp.float32(-10000.0)
            a0 = jnp.where(a0 == 0.0, neg, a0)
            a1 = jnp.where(a1 == 0.0, neg, a1)
            a0 = jnp.where(a0 >= 0.0, a0, 0.01 * a0)
            a1 = jnp.where(a1 >= 0.0, a1, 0.01 * a1)

            mx = jnp.maximum(a0, a1)
            red_v[pl.ds(L, L)] = jnp.full((L,), neg * 10.0, jnp.float32)
            for sh in (8, 4, 2, 1):
                red_v[pl.ds(0, L)] = mx
                mx = jnp.maximum(mx, red_v[pl.ds(sh, L)])
            amax = mx[0]

            e0 = jnp.exp(a0 - amax)
            e1 = jnp.exp(a1 - amax)
            inv = jnp.float32(1.0) / jnp.full((L,), _lane_sum(e0 + e1),
                                              jnp.float32)
            p0 = e0 * inv
            p1 = e1 * inv
            unif = jnp.float32(1.0 / M)
            p0 = jnp.where(p0 == unif, 0.0, p0)
            p1 = jnp.where(p1 == unif, 0.0, p1)

            cacc = [zeros for _ in range(NC)]
            for m in range(M):
                pv = p0 if m < L else p1
                pm = jnp.full((L,), pv[m % L], jnp.float32)
                for i in range(NC):
                    cacc[i] = cacc[i] + pm * v_v[pl.ds(m * D + i * L, L)]
            for i in range(NC):
                ctx_v[pl.ds(i * L, L)] = cacc[i]
            pltpu.sync_copy(ctx_v, out_hbm.at[pl.ds(r_loc * D, D)])
            return carry

        lax.fori_loop(0, RPW, row_body, 0)

    return sc_ctx


def kernel(input_ent, q, k, v, Wq, bq, Wk, Wv):
    B, N, M, D = k.shape
    QD = q.shape[1]
    del input_ent
    B_TC = B - _B_SC

    qk = pl.pallas_call(
        _qk_body,
        out_shape=jax.ShapeDtypeStruct((B, D), jnp.float32),
        in_specs=[
            pl.BlockSpec((B, QD), lambda: (0, 0)),
            pl.BlockSpec((D, QD), lambda: (0, 0)),
            pl.BlockSpec((1, D), lambda: (0, 0)),
            pl.BlockSpec((D, D), lambda: (0, 0)),
        ],
        out_specs=pl.BlockSpec((B, D), lambda: (0, 0)),
    )(q, Wq, bq.reshape(1, D), Wk)

    ctx_sc = _make_sc_ctx(B, N, M, D, B_TC)(
        k.reshape(B * N * M * D), v.reshape(B * N * M * D), qk.reshape(B * D))

    out_tc = pl.pallas_call(
        _attn_body,
        grid=(B_TC,),
        out_shape=jax.ShapeDtypeStruct((B_TC, N, D), jnp.float32),
        in_specs=[
            pl.BlockSpec((1, 1, D), lambda b: (b, 0, 0)),
            pl.BlockSpec((D, D), lambda b: (0, 0)),
            pl.BlockSpec((1, N, M, D), lambda b: (b, 0, 0, 0)),
            pl.BlockSpec((1, N, M, D), lambda b: (b, 0, 0, 0)),
        ],
        out_specs=pl.BlockSpec((1, N, D), lambda b: (b, 0, 0)),
        compiler_params=pltpu.CompilerParams(
            dimension_semantics=("arbitrary",),
        ),
    )(qk.reshape(B, 1, D), Wv, k, v)

    R = _B_SC * N
    out_sc = pl.pallas_call(
        _sc_proj_body,
        out_shape=jax.ShapeDtypeStruct((R, D), jnp.float32),
        in_specs=[
            pl.BlockSpec((R, D), lambda: (0, 0)),
            pl.BlockSpec((D, D), lambda: (0, 0)),
        ],
        out_specs=pl.BlockSpec((R, D), lambda: (0, 0)),
    )(ctx_sc.reshape(R, D), Wv)

    return jnp.concatenate([out_tc, out_sc.reshape(_B_SC, N, D)], axis=0)


# speedup vs baseline: 2.0537x; 2.0537x over previous
import functools
import math

import jax
import jax.numpy as jnp
from jax import lax
from jax.experimental import pallas as pl
from jax.experimental.pallas import tpu as pltpu
from jax.experimental.pallas import tpu_sc as plsc

_B_SC = 8


def _qk_body(q_ref, wq_ref, bq_ref, wk_ref, qk_ref):
    qwq = lax.dot_general(q_ref[...], wq_ref[...],
                          (((1,), (1,)), ((), ())),
                          preferred_element_type=jnp.float32)
    Q = jnp.tanh(qwq + bq_ref[...])
    qk_ref[...] = lax.dot_general(Q, wk_ref[...],
                                  (((1,), (0,)), ((), ())),
                                  preferred_element_type=jnp.float32)


def _attn_body(qk_ref, wv_ref, k_ref, v_ref, out_ref):
    kb = k_ref[0]
    vb = v_ref[0]
    qk = qk_ref[0]
    D = kb.shape[-1]
    M = kb.shape[1]
    scale = 1.0 / math.sqrt(D)

    att = jnp.sum(kb * qk[None, :, :], axis=2) * scale
    att = att.T
    att = jnp.where(att == 0.0, jnp.float32(-10000.0), att)
    att = jnp.where(att >= 0.0, att, 0.01 * att)
    amax = jnp.max(att, axis=0, keepdims=True)
    e = jnp.exp(att - amax)
    p = e / jnp.sum(e, axis=0, keepdims=True)
    p = jnp.where(p == jnp.float32(1.0 / M), jnp.float32(0.0), p)

    ctx = lax.dot_general(p, vb, (((0,), (1,)), ((1,), (0,))),
                          preferred_element_type=jnp.float32)
    out_ref[0] = lax.dot_general(ctx, wv_ref[...],
                                 (((1,), (1,)), ((), ())),
                                 preferred_element_type=jnp.float32)


def _sc_proj_body(ctx_ref, wv_ref, out_ref):
    out_ref[...] = lax.dot_general(ctx_ref[...], wv_ref[...],
                                   (((1,), (1,)), ((), ())),
                                   preferred_element_type=jnp.float32)


def _make_sc_ctx(B, N, M, D, b_start):
    B_SC = B - b_start
    R = B_SC * N
    MD = M * D
    W = 32
    RPW = R // W
    WPB = N // RPW
    L = 16
    NC = D // L
    scale = 1.0 / math.sqrt(D)
    mesh = plsc.VectorSubcoreMesh(core_axis_name="c", subcore_axis_name="s")

    def _tiled(m, i):
        return m, i * L

    @functools.partial(
        pl.kernel,
        out_type=jax.ShapeDtypeStruct((R * D,), jnp.float32),
        mesh=mesh,
        compiler_params=pltpu.CompilerParams(use_tc_tiling_on_sc=True),
        scratch_types=[
            pltpu.VMEM((D,), jnp.float32),
            pltpu.VMEM((M, D), jnp.float32),
            pltpu.VMEM((M, D), jnp.float32),
            pltpu.VMEM((2 * L,), jnp.float32),
            pltpu.VMEM((D,), jnp.float32),
            pltpu.SemaphoreType.DMA,
            pltpu.SemaphoreType.DMA,
        ],
    )
    def sc_ctx(k_hbm, v_hbm, qk_hbm, out_hbm,
               qk_v, k_v, v_v, red_v, ctx_v, sem_k, sem_v):
        wid = lax.axis_index("s") * 2 + lax.axis_index("c")
        b_loc = wid // WPB
        base_loc = b_loc * N + (wid % WPB) * RPW
        pltpu.sync_copy(qk_hbm.at[pl.ds((b_start + b_loc) * D, D)], qk_v)
        qkc = [qk_v[pl.ds(i * L, L)] for i in range(NC)]
        lane = lax.iota(jnp.int32, L)
        zeros = jnp.zeros((L,), jnp.float32)

        def _lane_sum(x):
            red_v[pl.ds(L, L)] = zeros
            for sh in (8, 4, 2, 1):
                red_v[pl.ds(0, L)] = x
                x = x + red_v[pl.ds(sh, L)]
            return x[0]

        def row_body(j, carry):
            r_loc = base_loc + j
            n_loc = (wid % WPB) * RPW + j
            b_glob = b_start + b_loc
            ck = pltpu.async_copy(k_hbm.at[b_glob, n_loc], k_v, sem_k)
            cv = pltpu.async_copy(v_hbm.at[b_glob, n_loc], v_v, sem_v)
            ck.wait()
            cv.wait()

            a0 = zeros
            a1 = zeros
            for m in range(M):
                r0, c0 = _tiled(m, 0)
                acc = k_v[r0, pl.ds(c0, L)] * qkc[0]
                for i in range(1, NC):
                    ri, ci = _tiled(m, i)
                    acc = acc + k_v[ri, pl.ds(ci, L)] * qkc[i]
                tot = jnp.full((L,), _lane_sum(acc) * scale, jnp.float32)
                if m < L:
                    a0 = jnp.where(lane == m, tot, a0)
                else:
                    a1 = jnp.where(lane == (m - L), tot, a1)

            neg = jnp.float32(-10000.0)
            a0 = jnp.where(a0 == 0.0, neg, a0)
            a1 = jnp.where(a1 == 0.0, neg, a1)
            a0 = jnp.where(a0 >= 0.0, a0, 0.01 * a0)
            a1 = jnp.where(a1 >= 0.0, a1, 0.01 * a1)

            mx = jnp.maximum(a0, a1)
            red_v[pl.ds(L, L)] = jnp.full((L,), neg * 10.0, jnp.float32)
            for sh in (8, 4, 2, 1):
                red_v[pl.ds(0, L)] = mx
                mx = jnp.maximum(mx, red_v[pl.ds(sh, L)])
            amax = mx[0]

            e0 = jnp.exp(a0 - amax)
            e1 = jnp.exp(a1 - amax)
            inv = jnp.float32(1.0) / jnp.full((L,), _lane_sum(e0 + e1),
                                              jnp.float32)
            p0 = e0 * inv
            p1 = e1 * inv
            unif = jnp.float32(1.0 / M)
            p0 = jnp.where(p0 == unif, 0.0, p0)
            p1 = jnp.where(p1 == unif, 0.0, p1)

            cacc = [zeros for _ in range(NC)]
            for m in range(M):
                pv = p0 if m < L else p1
                pm = jnp.full((L,), pv[m % L], jnp.float32)
                for i in range(NC):
                    ri, ci = _tiled(m, i)
                    cacc[i] = cacc[i] + pm * v_v[ri, pl.ds(ci, L)]
            for i in range(NC):
                ctx_v[pl.ds(i * L, L)] = cacc[i]
            pltpu.sync_copy(ctx_v, out_hbm.at[pl.ds(r_loc * D, D)])
            return carry

        lax.fori_loop(0, RPW, row_body, 0)

    return sc_ctx


def kernel(input_ent, q, k, v, Wq, bq, Wk, Wv):
    B, N, M, D = k.shape
    QD = q.shape[1]
    del input_ent
    B_TC = B - _B_SC

    qk = pl.pallas_call(
        _qk_body,
        out_shape=jax.ShapeDtypeStruct((B, D), jnp.float32),
        in_specs=[
            pl.BlockSpec((B, QD), lambda: (0, 0)),
            pl.BlockSpec((D, QD), lambda: (0, 0)),
            pl.BlockSpec((1, D), lambda: (0, 0)),
            pl.BlockSpec((D, D), lambda: (0, 0)),
        ],
        out_specs=pl.BlockSpec((B, D), lambda: (0, 0)),
    )(q, Wq, bq.reshape(1, D), Wk)

    ctx_sc = _make_sc_ctx(B, N, M, D, B_TC)(k, v, qk.reshape(B * D))

    out_tc = pl.pallas_call(
        _attn_body,
        grid=(B_TC,),
        out_shape=jax.ShapeDtypeStruct((B_TC, N, D), jnp.float32),
        in_specs=[
            pl.BlockSpec((1, 1, D), lambda b: (b, 0, 0)),
            pl.BlockSpec((D, D), lambda b: (0, 0)),
            pl.BlockSpec((1, N, M, D), lambda b: (b, 0, 0, 0)),
            pl.BlockSpec((1, N, M, D), lambda b: (b, 0, 0, 0)),
        ],
        out_specs=pl.BlockSpec((1, N, D), lambda b: (b, 0, 0)),
        compiler_params=pltpu.CompilerParams(
            dimension_semantics=("arbitrary",),
        ),
    )(qk.reshape(B, 1, D), Wv, k, v)

    R = _B_SC * N
    out_sc = pl.pallas_call(
        _sc_proj_body,
        out_shape=jax.ShapeDtypeStruct((R, D), jnp.float32),
        in_specs=[
            pl.BlockSpec((R, D), lambda: (0, 0)),
            pl.BlockSpec((D, D), lambda: (0, 0)),
        ],
        out_specs=pl.BlockSpec((R, D), lambda: (0, 0)),
    )(ctx_sc.reshape(R, D), Wv)

    return jnp.concatenate([out_tc, out_sc.reshape(_B_SC, N, D)], axis=0)


# speedup vs baseline: 2.6020x; 1.2670x over previous
import functools
import math

import jax
import jax.numpy as jnp
from jax import lax
from jax.experimental import pallas as pl
from jax.experimental.pallas import tpu as pltpu
from jax.experimental.pallas import tpu_sc as plsc

_B_SC = 4


def _qk_body(q_ref, wq_ref, bq_ref, wk_ref, qk_ref):
    qwq = lax.dot_general(q_ref[...], wq_ref[...],
                          (((1,), (1,)), ((), ())),
                          preferred_element_type=jnp.float32)
    Q = jnp.tanh(qwq + bq_ref[...])
    qk_ref[...] = lax.dot_general(Q, wk_ref[...],
                                  (((1,), (0,)), ((), ())),
                                  preferred_element_type=jnp.float32)


def _attn_body(qk_ref, wv_ref, k_ref, v_ref, out_ref):
    kb = k_ref[0]
    vb = v_ref[0]
    qk = qk_ref[0]
    D = kb.shape[-1]
    M = kb.shape[1]
    scale = 1.0 / math.sqrt(D)

    att = jnp.sum(kb * qk[None, :, :], axis=2) * scale
    att = att.T
    att = jnp.where(att == 0.0, jnp.float32(-10000.0), att)
    att = jnp.where(att >= 0.0, att, 0.01 * att)
    amax = jnp.max(att, axis=0, keepdims=True)
    e = jnp.exp(att - amax)
    p = e / jnp.sum(e, axis=0, keepdims=True)
    p = jnp.where(p == jnp.float32(1.0 / M), jnp.float32(0.0), p)

    ctx = lax.dot_general(p, vb, (((0,), (1,)), ((1,), (0,))),
                          preferred_element_type=jnp.float32)
    out_ref[0] = lax.dot_general(ctx, wv_ref[...],
                                 (((1,), (1,)), ((), ())),
                                 preferred_element_type=jnp.float32)


def _sc_proj_body(ctx_ref, wv_ref, out_ref):
    out_ref[...] = lax.dot_general(ctx_ref[...], wv_ref[...],
                                   (((1,), (1,)), ((), ())),
                                   preferred_element_type=jnp.float32)


def _make_sc_ctx(B, N, M, D, b_start):
    B_SC = B - b_start
    R = B_SC * N
    MD = M * D
    W = 32
    RPW = R // W
    WPB = N // RPW
    L = 16
    NC = D // L
    scale = 1.0 / math.sqrt(D)
    mesh = plsc.VectorSubcoreMesh(core_axis_name="c", subcore_axis_name="s")

    def _tiled(m, i):
        return m, i * L

    @functools.partial(
        pl.kernel,
        out_type=jax.ShapeDtypeStruct((R * D,), jnp.float32),
        mesh=mesh,
        compiler_params=pltpu.CompilerParams(use_tc_tiling_on_sc=True),
        scratch_types=[
            pltpu.VMEM((D,), jnp.float32),
            pltpu.VMEM((M, D), jnp.float32),
            pltpu.VMEM((M, D), jnp.float32),
            pltpu.VMEM((2 * L,), jnp.float32),
            pltpu.VMEM((D,), jnp.float32),
            pltpu.SemaphoreType.DMA,
            pltpu.SemaphoreType.DMA,
        ],
    )
    def sc_ctx(k_hbm, v_hbm, qk_hbm, out_hbm,
               qk_v, k_v, v_v, red_v, ctx_v, sem_k, sem_v):
        wid = lax.axis_index("s") * 2 + lax.axis_index("c")
        b_loc = wid // WPB
        base_loc = b_loc * N + (wid % WPB) * RPW
        pltpu.sync_copy(qk_hbm.at[pl.ds((b_start + b_loc) * D, D)], qk_v)
        qkc = [qk_v[pl.ds(i * L, L)] for i in range(NC)]
        lane = lax.iota(jnp.int32, L)
        zeros = jnp.zeros((L,), jnp.float32)

        def _lane_sum(x):
            red_v[pl.ds(L, L)] = zeros
            for sh in (8, 4, 2, 1):
                red_v[pl.ds(0, L)] = x
                x = x + red_v[pl.ds(sh, L)]
            return x[0]

        def row_body(j, carry):
            r_loc = base_loc + j
            n_loc = (wid % WPB) * RPW + j
            b_glob = b_start + b_loc
            ck = pltpu.async_copy(k_hbm.at[b_glob, n_loc], k_v, sem_k)
            cv = pltpu.async_copy(v_hbm.at[b_glob, n_loc], v_v, sem_v)
            ck.wait()
            cv.wait()

            a0 = zeros
            a1 = zeros
            for m in range(M):
                r0, c0 = _tiled(m, 0)
                acc = k_v[r0, pl.ds(c0, L)] * qkc[0]
                for i in range(1, NC):
                    ri, ci = _tiled(m, i)
                    acc = acc + k_v[ri, pl.ds(ci, L)] * qkc[i]
                tot = jnp.full((L,), _lane_sum(acc) * scale, jnp.float32)
                if m < L:
                    a0 = jnp.where(lane == m, tot, a0)
                else:
                    a1 = jnp.where(lane == (m - L), tot, a1)

            neg = jnp.float32(-10000.0)
            a0 = jnp.where(a0 == 0.0, neg, a0)
            a1 = jnp.where(a1 == 0.0, neg, a1)
            a0 = jnp.where(a0 >= 0.0, a0, 0.01 * a0)
            a1 = jnp.where(a1 >= 0.0, a1, 0.01 * a1)

            mx = jnp.maximum(a0, a1)
            red_v[pl.ds(L, L)] = jnp.full((L,), neg * 10.0, jnp.float32)
            for sh in (8, 4, 2, 1):
                red_v[pl.ds(0, L)] = mx
                mx = jnp.maximum(mx, red_v[pl.ds(sh, L)])
            amax = mx[0]

            e0 = jnp.exp(a0 - amax)
            e1 = jnp.exp(a1 - amax)
            inv = jnp.float32(1.0) / jnp.full((L,), _lane_sum(e0 + e1),
                                              jnp.float32)
            p0 = e0 * inv
            p1 = e1 * inv
            unif = jnp.float32(1.0 / M)
            p0 = jnp.where(p0 == unif, 0.0, p0)
            p1 = jnp.where(p1 == unif, 0.0, p1)

            cacc = [zeros for _ in range(NC)]
            for m in range(M):
                pv = p0 if m < L else p1
                pm = jnp.full((L,), pv[m % L], jnp.float32)
                for i in range(NC):
                    ri, ci = _tiled(m, i)
                    cacc[i] = cacc[i] + pm * v_v[ri, pl.ds(ci, L)]
            for i in range(NC):
                ctx_v[pl.ds(i * L, L)] = cacc[i]
            pltpu.sync_copy(ctx_v, out_hbm.at[pl.ds(r_loc * D, D)])
            return carry

        lax.fori_loop(0, RPW, row_body, 0)

    return sc_ctx


def kernel(input_ent, q, k, v, Wq, bq, Wk, Wv):
    B, N, M, D = k.shape
    QD = q.shape[1]
    del input_ent
    B_TC = B - _B_SC

    qk = pl.pallas_call(
        _qk_body,
        out_shape=jax.ShapeDtypeStruct((B, D), jnp.float32),
        in_specs=[
            pl.BlockSpec((B, QD), lambda: (0, 0)),
            pl.BlockSpec((D, QD), lambda: (0, 0)),
            pl.BlockSpec((1, D), lambda: (0, 0)),
            pl.BlockSpec((D, D), lambda: (0, 0)),
        ],
        out_specs=pl.BlockSpec((B, D), lambda: (0, 0)),
    )(q, Wq, bq.reshape(1, D), Wk)

    ctx_sc = _make_sc_ctx(B, N, M, D, B_TC)(k, v, qk.reshape(B * D))

    out_tc = pl.pallas_call(
        _attn_body,
        grid=(B_TC,),
        out_shape=jax.ShapeDtypeStruct((B_TC, N, D), jnp.float32),
        in_specs=[
            pl.BlockSpec((1, 1, D), lambda b: (b, 0, 0)),
            pl.BlockSpec((D, D), lambda b: (0, 0)),
            pl.BlockSpec((1, N, M, D), lambda b: (b, 0, 0, 0)),
            pl.BlockSpec((1, N, M, D), lambda b: (b, 0, 0, 0)),
        ],
        out_specs=pl.BlockSpec((1, N, D), lambda b: (b, 0, 0)),
        compiler_params=pltpu.CompilerParams(
            dimension_semantics=("arbitrary",),
        ),
    )(qk.reshape(B, 1, D), Wv, k, v)

    R = _B_SC * N
    out_sc = pl.pallas_call(
        _sc_proj_body,
        out_shape=jax.ShapeDtypeStruct((R, D), jnp.float32),
        in_specs=[
            pl.BlockSpec((R, D), lambda: (0, 0)),
            pl.BlockSpec((D, D), lambda: (0, 0)),
        ],
        out_specs=pl.BlockSpec((R, D), lambda: (0, 0)),
    )(ctx_sc.reshape(R, D), Wv)

    return jnp.concatenate([out_tc, out_sc.reshape(_B_SC, N, D)], axis=0)
